# baseline (speedup 1.0000x reference)
import jax
import jax.numpy as jnp
from jax import lax
from jax.experimental import pallas as pl
from jax.experimental.pallas import tpu as pltpu
from jax.experimental.pallas import tpu_sc as plsc

N = 10000
E = 160000
D = 256
NC = 2
NS = 16
NT = NC * NS
ROWS = 320
TRASH = ROWS
WIN = 2000
NWIN = E // WIN
CH = 128
CCAP = 2304


def _body(nf_hbm, src_hbm, dst_hbm, len_hbm, cut_hbm, par_hbm, out_hbm,
          acc, srcw, dstw, csrc, cdst, ceid, rows0, lensel, cutsel, wsel,
          psrc, peid, pdst, parv, gsem):
    c = lax.axis_index("c")
    s = lax.axis_index("s")
    t = c * NS + s
    gbase = t * ROWS
    rlimit = jnp.minimum(ROWS, N - gbase)

    pltpu.sync_copy(par_hbm, parv)
    zeros16 = jnp.zeros((16,), jnp.int32)
    lanes = lax.iota(jnp.int32, 16)
    mc_v = parv[pl.ds(0, 16)]
    pf_v = parv[pl.ds(16, 16)]
    al_v = parv[pl.ds(32, 16)]

    @pl.loop(0, rlimit // 16)
    def _init(k):
        r0 = k * 16
        pltpu.sync_copy(nf_hbm.at[pl.ds(gbase + r0, 16)],
                        acc.at[pl.ds(r0, 16)])

    @pl.loop(0, rlimit)
    def _init_scale(r):
        for m in range(D // 16):
            sl = pl.ds(16 * m, 16)
            acc[r, sl] = acc[r, sl] * mc_v

    def _fire(b):
        for j in range(CH // 16):
            sl = pl.ds(16 * j, 16)
            psrc[sl] = csrc[pl.ds(b + 16 * j, 16)]
            peid[sl] = ceid[pl.ds(b + 16 * j, 16)]
            pdst[sl] = cdst[pl.ds(b + 16 * j, 16)]
        pltpu.async_copy(nf_hbm.at[psrc], rows0, gsem)
        pltpu.async_copy(len_hbm.at[peid], lensel, gsem)
        pltpu.async_copy(cut_hbm.at[peid], cutsel, gsem)

    def _finish():
        pltpu.make_async_copy(nf_hbm.at[psrc], rows0, gsem).wait()
        pltpu.make_async_copy(len_hbm.at[peid], lensel, gsem).wait()
        pltpu.make_async_copy(cut_hbm.at[peid], cutsel, gsem).wait()

        for j in range(CH // 16):
            sl = pl.ds(16 * j, 16)
            wsel[sl] = pf_v * jnp.exp(-(al_v * lensel[sl])) * cutsel[sl]

        @pl.loop(0, CH // 16)
        def _accum(g):
            base = 16 * g
            dv = pdst[pl.ds(base, 16)]
            wv = wsel[pl.ds(base, 16)]
            for j in range(16):
                dloc = dv[j]
                w_j = wv[j]
                for m in range(D // 16):
                    sl = pl.ds(16 * m, 16)
                    plsc.addupdate(acc.at[dloc, sl], rows0[base + j, sl] * w_j)

    @pl.loop(0, NWIN, init_carry=(jnp.int32(0), jnp.int32(0)))
    def _window(wi, carry):
        ptr, pend = carry
        wb = wi * WIN
        pltpu.sync_copy(src_hbm.at[pl.ds(wb, WIN)], srcw)
        pltpu.sync_copy(dst_hbm.at[pl.ds(wb, WIN)], dstw)

        def _cstep(b, p):
            d16 = dstw[pl.ds(b, 16)]
            s16 = srcw[pl.ds(b, 16)]
            dloc = d16 - gbase
            keep = (dloc >= 0) & (dloc < rlimit)
            win16 = pl.ds(p, 16)
            plsc.store_compressed(csrc.at[win16], s16, mask=keep)
            plsc.store_compressed(cdst.at[win16], dloc, mask=keep)
            plsc.store_compressed(ceid.at[win16], wb + b + lanes, mask=keep)
            return p + plsc.all_reduce_population_count(keep)[0]

        @pl.loop(0, WIN // 32, init_carry=ptr)
        def _compact(i, p):
            b = i * 32
            p = _cstep(b, p)
            return _cstep(b + 16, p)

        ptr2 = _cstep(WIN - 16, _compact)
        full = ptr2 // CH

        @pl.loop(0, full, init_carry=pend)
        def _chunks(k, pnd):
            @pl.when(pnd == 1)
            def _():
                _finish()

            _fire(k * CH)
            return jnp.int32(1)

        pend2 = _chunks

        tb = full * CH
        for j in range(CH // 16):
            sl_dst = pl.ds(16 * j, 16)
            sl_src = pl.ds(tb + 16 * j, 16)
            csrc[sl_dst] = csrc[sl_src]
            cdst[sl_dst] = cdst[sl_src]
            ceid[sl_dst] = ceid[sl_src]
        return (ptr2 - tb, pend2)

    nrem, pend_f = _window

    @pl.when(pend_f == 1)
    def _drain_pending():
        _finish()

    @pl.when(nrem > 0)
    def _final():
        j0 = (nrem // 16) * 16
        bkeep = lanes < (nrem - j0)
        bw = pl.ds(j0, 16)
        csrc[bw] = jnp.where(bkeep, csrc[bw], 0)
        cdst[bw] = jnp.where(bkeep, cdst[bw], TRASH)
        ceid[bw] = jnp.where(bkeep, ceid[bw], 0)

        @pl.loop(j0 + 16, CH, step=16)
        def _pad(j):
            jw = pl.ds(j, 16)
            csrc[jw] = zeros16
            cdst[jw] = zeros16 + TRASH
            ceid[jw] = zeros16

        _fire(0)
        _finish()

    @pl.loop(0, rlimit // 16)
    def _drain(k):
        r0 = k * 16
        pltpu.sync_copy(acc.at[pl.ds(r0, 16)],
                        out_hbm.at[pl.ds(gbase + r0, 16)])


@jax.jit
def _interaction(nf, src, dst, lens, cuts, par):
    mesh = plsc.VectorSubcoreMesh(core_axis_name="c", subcore_axis_name="s",
                                  num_cores=NC, num_subcores=NS)
    f = pl.kernel(
        _body,
        out_type=jax.ShapeDtypeStruct((N, D), jnp.float32),
        mesh=mesh,
        compiler_params=pltpu.CompilerParams(needs_layout_passes=False,
                                             use_tc_tiling_on_sc=False),
        scratch_types=[
            pltpu.VMEM((ROWS + 8, D), jnp.float32),
            pltpu.VMEM((WIN,), jnp.int32),
            pltpu.VMEM((WIN,), jnp.int32),
            pltpu.VMEM((CCAP,), jnp.int32),
            pltpu.VMEM((CCAP,), jnp.int32),
            pltpu.VMEM((CCAP,), jnp.int32),
            pltpu.VMEM((CH, D), jnp.float32),
            pltpu.VMEM((CH,), jnp.float32),
            pltpu.VMEM((CH,), jnp.float32),
            pltpu.VMEM((CH,), jnp.float32),
            pltpu.VMEM((CH,), jnp.int32),
            pltpu.VMEM((CH,), jnp.int32),
            pltpu.VMEM((CH,), jnp.int32),
            pltpu.VMEM((48,), jnp.float32),
            pltpu.SemaphoreType.DMA,
        ],
    )
    return f(nf, src, dst, lens, cuts, par)


def kernel(node_feat, edge_lengths, radial_cutoff, edge_index, n_nodes,
           memory_coef, rbf_prefactor, rbf_alpha):
    nf = node_feat.reshape(N, D)
    src = edge_index[0].astype(jnp.int32)
    dst = edge_index[1].astype(jnp.int32)
    lens = edge_lengths.reshape(E)
    cuts = radial_cutoff.reshape(E)
    par = jnp.concatenate([
        jnp.full((16,), jnp.asarray(memory_coef, jnp.float32)),
        jnp.full((16,), jnp.asarray(rbf_prefactor, jnp.float32)),
        jnp.full((16,), jnp.asarray(rbf_alpha, jnp.float32)),
    ])
    out = _interaction(nf, src, dst, lens, cuts, par)
    return out.reshape(node_feat.shape)


# speedup vs baseline: 1.0158x; 1.0158x over previous
import jax
import jax.numpy as jnp
from jax import lax
from jax.experimental import pallas as pl
from jax.experimental.pallas import tpu as pltpu
from jax.experimental.pallas import tpu_sc as plsc

N = 10000
E = 160000
D = 256
NC = 2
NS = 16
NT = NC * NS
ROWS = 320
TRASH = ROWS
WIN = 2000
NWIN = E // WIN
CH = 128
CCAP = 2304


def _body(nf_hbm, src_hbm, dst_hbm, len_hbm, cut_hbm, par_hbm, out_hbm,
          acc, srcw, dstw, csrc, cpak, rows0, lensel, cutsel, wsel,
          psrc, peid, pdst, parv, gsem):
    c = lax.axis_index("c")
    s = lax.axis_index("s")
    t = c * NS + s
    gbase = t * ROWS
    rlimit = jnp.minimum(ROWS, N - gbase)

    pltpu.sync_copy(par_hbm, parv)
    zeros16 = jnp.zeros((16,), jnp.int32)
    lanes = lax.iota(jnp.int32, 16)
    mc_v = parv[pl.ds(0, 16)]
    pf_v = parv[pl.ds(16, 16)]
    al_v = parv[pl.ds(32, 16)]

    @pl.loop(0, rlimit // 16)
    def _init(k):
        r0 = k * 16
        pltpu.sync_copy(nf_hbm.at[pl.ds(gbase + r0, 16)],
                        acc.at[pl.ds(r0, 16)])

    @pl.loop(0, rlimit)
    def _init_scale(r):
        for m in range(D // 16):
            sl = pl.ds(16 * m, 16)
            acc[r, sl] = acc[r, sl] * mc_v

    def _fire(b):
        for j in range(CH // 16):
            sl = pl.ds(16 * j, 16)
            psrc[sl] = csrc[pl.ds(b + 16 * j, 16)]
            pk = cpak[pl.ds(b + 16 * j, 16)]
            peid[sl] = pk & 0x3FFFF
            pdst[sl] = lax.shift_right_logical(pk, 18)
        pltpu.async_copy(nf_hbm.at[psrc], rows0, gsem)
        pltpu.async_copy(len_hbm.at[peid], lensel, gsem)
        pltpu.async_copy(cut_hbm.at[peid], cutsel, gsem)

    def _finish():
        pltpu.make_async_copy(nf_hbm.at[psrc], rows0, gsem).wait()
        pltpu.make_async_copy(len_hbm.at[peid], lensel, gsem).wait()
        pltpu.make_async_copy(cut_hbm.at[peid], cutsel, gsem).wait()

        for j in range(CH // 16):
            sl = pl.ds(16 * j, 16)
            wsel[sl] = pf_v * jnp.exp(-(al_v * lensel[sl])) * cutsel[sl]

        @pl.loop(0, CH // 16)
        def _accum(g):
            base = 16 * g
            dv = pdst[pl.ds(base, 16)]
            wv = wsel[pl.ds(base, 16)]
            for j in range(16):
                dloc = dv[j]
                w_j = wv[j]
                for m in range(D // 16):
                    sl = pl.ds(16 * m, 16)
                    plsc.addupdate(acc.at[dloc, sl], rows0[base + j, sl] * w_j)

    @pl.loop(0, NWIN, init_carry=(jnp.int32(0), jnp.int32(0)))
    def _window(wi, carry):
        ptr, pend = carry
        wb = wi * WIN
        pltpu.sync_copy(src_hbm.at[pl.ds(wb, WIN)], srcw)
        pltpu.sync_copy(dst_hbm.at[pl.ds(wb, WIN)], dstw)

        def _cstep(b, p):
            d16 = dstw[pl.ds(b, 16)]
            s16 = srcw[pl.ds(b, 16)]
            dloc = d16 - gbase
            keep = (dloc >= 0) & (dloc < rlimit)
            win16 = pl.ds(p, 16)
            packed = (dloc << 18) | (wb + b + lanes)
            plsc.store_compressed(csrc.at[win16], s16, mask=keep)
            plsc.store_compressed(cpak.at[win16], packed, mask=keep)
            return p + plsc.all_reduce_population_count(keep)[0]

        @pl.loop(0, WIN // 32, init_carry=ptr)
        def _compact(i, p):
            b = i * 32
            p = _cstep(b, p)
            return _cstep(b + 16, p)

        ptr2 = _cstep(WIN - 16, _compact)
        full = ptr2 // CH

        @pl.loop(0, full, init_carry=pend)
        def _chunks(k, pnd):
            @pl.when(pnd == 1)
            def _():
                _finish()

            _fire(k * CH)
            return jnp.int32(1)

        pend2 = _chunks

        tb = full * CH
        for j in range(CH // 16):
            sl_dst = pl.ds(16 * j, 16)
            sl_src = pl.ds(tb + 16 * j, 16)
            csrc[sl_dst] = csrc[sl_src]
            cpak[sl_dst] = cpak[sl_src]
        return (ptr2 - tb, pend2)

    nrem, pend_f = _window

    @pl.when(pend_f == 1)
    def _drain_pending():
        _finish()

    @pl.when(nrem > 0)
    def _final():
        j0 = (nrem // 16) * 16
        bkeep = lanes < (nrem - j0)
        bw = pl.ds(j0, 16)
        csrc[bw] = jnp.where(bkeep, csrc[bw], 0)
        cpak[bw] = jnp.where(bkeep, cpak[bw], TRASH << 18)

        @pl.loop(j0 + 16, CH, step=16)
        def _pad(j):
            jw = pl.ds(j, 16)
            csrc[jw] = zeros16
            cpak[jw] = zeros16 + (TRASH << 18)

        _fire(0)
        _finish()

    @pl.loop(0, rlimit // 16)
    def _drain(k):
        r0 = k * 16
        pltpu.sync_copy(acc.at[pl.ds(r0, 16)],
                        out_hbm.at[pl.ds(gbase + r0, 16)])


@jax.jit
def _interaction(nf, src, dst, lens, cuts, par):
    mesh = plsc.VectorSubcoreMesh(core_axis_name="c", subcore_axis_name="s",
                                  num_cores=NC, num_subcores=NS)
    f = pl.kernel(
        _body,
        out_type=jax.ShapeDtypeStruct((N, D), jnp.float32),
        mesh=mesh,
        compiler_params=pltpu.CompilerParams(needs_layout_passes=False,
                                             use_tc_tiling_on_sc=False),
        scratch_types=[
            pltpu.VMEM((ROWS + 8, D), jnp.float32),
            pltpu.VMEM((WIN,), jnp.int32),
            pltpu.VMEM((WIN,), jnp.int32),
            pltpu.VMEM((CCAP,), jnp.int32),
            pltpu.VMEM((CCAP,), jnp.int32),
            pltpu.VMEM((CH, D), jnp.float32),
            pltpu.VMEM((CH,), jnp.float32),
            pltpu.VMEM((CH,), jnp.float32),
            pltpu.VMEM((CH,), jnp.float32),
            pltpu.VMEM((CH,), jnp.int32),
            pltpu.VMEM((CH,), jnp.int32),
            pltpu.VMEM((CH,), jnp.int32),
            pltpu.VMEM((48,), jnp.float32),
            pltpu.SemaphoreType.DMA,
        ],
    )
    return f(nf, src, dst, lens, cuts, par)


def kernel(node_feat, edge_lengths, radial_cutoff, edge_index, n_nodes,
           memory_coef, rbf_prefactor, rbf_alpha):
    nf = node_feat.reshape(N, D)
    src = edge_index[0].astype(jnp.int32)
    dst = edge_index[1].astype(jnp.int32)
    lens = edge_lengths.reshape(E)
    cuts = radial_cutoff.reshape(E)
    par = jnp.concatenate([
        jnp.full((16,), jnp.asarray(memory_coef, jnp.float32)),
        jnp.full((16,), jnp.asarray(rbf_prefactor, jnp.float32)),
        jnp.full((16,), jnp.asarray(rbf_alpha, jnp.float32)),
    ])
    out = _interaction(nf, src, dst, lens, cuts, par)
    return out.reshape(node_feat.shape)


# speedup vs baseline: 1.0804x; 1.0637x over previous
import jax
import jax.numpy as jnp
from jax import lax
from jax.experimental import pallas as pl
from jax.experimental.pallas import tpu as pltpu
from jax.experimental.pallas import tpu_sc as plsc

N = 10000
E = 160000
D = 256
NC = 2
NS = 16
NT = NC * NS
ROWS = 320
TRASH = ROWS
WIN = 2000
NWIN = E // WIN
CH = 128
CCAP = 2304


def _body(nf_hbm, ei_hbm, len_hbm, cut_hbm, par_hbm, out_hbm,
          acc, ew, csrc, cpak, rows0, lensel, cutsel, wsel,
          psrc, peid, pdst, parv, gsem):
    c = lax.axis_index("c")
    s = lax.axis_index("s")
    t = c * NS + s
    gbase = t * ROWS
    rlimit = jnp.minimum(ROWS, N - gbase)

    pltpu.sync_copy(par_hbm, parv)
    zeros16 = jnp.zeros((16,), jnp.int32)
    lanes = lax.iota(jnp.int32, 16)
    mc_v = parv[pl.ds(0, 16)]
    pf_v = parv[pl.ds(16, 16)]
    al_v = parv[pl.ds(32, 16)]

    @pl.loop(0, rlimit // 16)
    def _init(k):
        r0 = k * 16
        pltpu.sync_copy(nf_hbm.at[pl.ds(gbase + r0, 16)],
                        acc.at[pl.ds(r0, 16)])

    @pl.loop(0, rlimit)
    def _init_scale(r):
        for m in range(D // 16):
            sl = pl.ds(16 * m, 16)
            acc[r, sl] = acc[r, sl] * mc_v

    def _fire(b):
        for j in range(CH // 16):
            sl = pl.ds(16 * j, 16)
            psrc[sl] = csrc[pl.ds(b + 16 * j, 16)]
            pk = cpak[pl.ds(b + 16 * j, 16)]
            peid[sl] = pk & 0x3FFFF
            pdst[sl] = lax.shift_right_logical(pk, 18)
        pltpu.async_copy(nf_hbm.at[psrc], rows0, gsem)
        pltpu.async_copy(len_hbm.at[peid], lensel, gsem)
        pltpu.async_copy(cut_hbm.at[peid], cutsel, gsem)

    def _finish():
        pltpu.make_async_copy(nf_hbm.at[psrc], rows0, gsem).wait()
        pltpu.make_async_copy(len_hbm.at[peid], lensel, gsem).wait()
        pltpu.make_async_copy(cut_hbm.at[peid], cutsel, gsem).wait()

        for j in range(CH // 16):
            sl = pl.ds(16 * j, 16)
            wsel[sl] = pf_v * jnp.exp(-(al_v * lensel[sl])) * cutsel[sl]

        @pl.loop(0, CH // 16)
        def _accum(g):
            base = 16 * g
            dv = pdst[pl.ds(base, 16)]
            wv = wsel[pl.ds(base, 16)]
            for j in range(16):
                dloc = dv[j]
                w_j = wv[j]
                for m in range(D // 16):
                    sl = pl.ds(16 * m, 16)
                    plsc.addupdate(acc.at[dloc, sl], rows0[base + j, sl] * w_j)

    @pl.loop(0, NWIN, init_carry=(jnp.int32(0), jnp.int32(0)))
    def _window(wi, carry):
        ptr, pend = carry
        wb = wi * WIN
        pltpu.sync_copy(ei_hbm.at[:, pl.ds(wb, WIN)], ew)

        def _cstep(b, p):
            d16 = ew[1, pl.ds(b, 16)]
            s16 = ew[0, pl.ds(b, 16)]
            dloc = d16 - gbase
            keep = (dloc >= 0) & (dloc < rlimit)
            win16 = pl.ds(p, 16)
            packed = (dloc << 18) | (wb + b + lanes)
            plsc.store_compressed(csrc.at[win16], s16, mask=keep)
            plsc.store_compressed(cpak.at[win16], packed, mask=keep)
            return p + plsc.all_reduce_population_count(keep)[0]

        @pl.loop(0, WIN // 32, init_carry=ptr)
        def _compact(i, p):
            b = i * 32
            p = _cstep(b, p)
            return _cstep(b + 16, p)

        ptr2 = _cstep(WIN - 16, _compact)
        full = ptr2 // CH

        @pl.loop(0, full, init_carry=pend)
        def _chunks(k, pnd):
            @pl.when(pnd == 1)
            def _():
                _finish()

            _fire(k * CH)
            return jnp.int32(1)

        pend2 = _chunks

        tb = full * CH
        for j in range(CH // 16):
            sl_dst = pl.ds(16 * j, 16)
            sl_src = pl.ds(tb + 16 * j, 16)
            csrc[sl_dst] = csrc[sl_src]
            cpak[sl_dst] = cpak[sl_src]
        return (ptr2 - tb, pend2)

    nrem, pend_f = _window

    @pl.when(pend_f == 1)
    def _drain_pending():
        _finish()

    @pl.when(nrem > 0)
    def _final():
        j0 = (nrem // 16) * 16
        bkeep = lanes < (nrem - j0)
        bw = pl.ds(j0, 16)
        csrc[bw] = jnp.where(bkeep, csrc[bw], 0)
        cpak[bw] = jnp.where(bkeep, cpak[bw], TRASH << 18)

        @pl.loop(j0 + 16, CH, step=16)
        def _pad(j):
            jw = pl.ds(j, 16)
            csrc[jw] = zeros16
            cpak[jw] = zeros16 + (TRASH << 18)

        _fire(0)
        _finish()

    @pl.loop(0, rlimit // 16)
    def _drain(k):
        r0 = k * 16
        pltpu.sync_copy(acc.at[pl.ds(r0, 16)],
                        out_hbm.at[pl.ds(gbase + r0, 16)])


@jax.jit
def _interaction(nf, ei, lens, cuts, par):
    mesh = plsc.VectorSubcoreMesh(core_axis_name="c", subcore_axis_name="s",
                                  num_cores=NC, num_subcores=NS)
    f = pl.kernel(
        _body,
        out_type=jax.ShapeDtypeStruct((N, D), jnp.float32),
        mesh=mesh,
        compiler_params=pltpu.CompilerParams(needs_layout_passes=False,
                                             use_tc_tiling_on_sc=False),
        scratch_types=[
            pltpu.VMEM((ROWS + 8, D), jnp.float32),
            pltpu.VMEM((2, WIN), jnp.int32),
            pltpu.VMEM((CCAP,), jnp.int32),
            pltpu.VMEM((CCAP,), jnp.int32),
            pltpu.VMEM((CH, D), jnp.float32),
            pltpu.VMEM((CH,), jnp.float32),
            pltpu.VMEM((CH,), jnp.float32),
            pltpu.VMEM((CH,), jnp.float32),
            pltpu.VMEM((CH,), jnp.int32),
            pltpu.VMEM((CH,), jnp.int32),
            pltpu.VMEM((CH,), jnp.int32),
            pltpu.VMEM((48,), jnp.float32),
            pltpu.SemaphoreType.DMA,
        ],
    )
    return f(nf, ei, lens, cuts, par)


def kernel(node_feat, edge_lengths, radial_cutoff, edge_index, n_nodes,
           memory_coef, rbf_prefactor, rbf_alpha):
    nf = node_feat.reshape(N, D)
    ei = edge_index.astype(jnp.int32)
    lens = edge_lengths.reshape(E)
    cuts = radial_cutoff.reshape(E)
    par = jnp.concatenate([
        jnp.full((16,), jnp.asarray(memory_coef, jnp.float32)),
        jnp.full((16,), jnp.asarray(rbf_prefactor, jnp.float32)),
        jnp.full((16,), jnp.asarray(rbf_alpha, jnp.float32)),
    ])
    out = _interaction(nf, ei, lens, cuts, par)
    return out.reshape(node_feat.shape)
